# baseline (speedup 1.0000x reference)
import functools

import jax
import jax.numpy as jnp
from jax import lax
from jax.experimental import pallas as pl
from jax.experimental.pallas import tpu as pltpu
from jax.experimental.pallas import tpu_sc as plsc

N = 10000
E = 320000
D = 128
BN_EPS = 1e-3

NC = 2
NS = 16
NW = NC * NS
CHUNK = 128
K = 8 * (-(-E // (NW * CHUNK * 8)))
EPW = K * CHUNK
EPAD = NW * EPW
NPAD = 10240
ZR = 16


def _sc_agg_body(
    h_hbm, srcr_hbm, dstr_hbm, out_hbm, sidx, didx, rows0, rows1, zbuf, acc, sem0, sem1
):
    c = lax.axis_index("c")
    s = lax.axis_index("s")
    w = (1 - c) * NS + s

    zeros16 = jnp.zeros((16,), jnp.float32)
    for r in range(ZR):
        for q in range(D // 16):
            zbuf[r, pl.ds(q * 16, 16)] = zeros16

    rows_per_tile = NPAD // NS

    def zero_body(t, carry):
        pltpu.sync_copy(zbuf, acc.at[pl.ds(s * rows_per_tile + t * ZR, ZR)])
        return carry

    lax.fori_loop(0, rows_per_tile // ZR, zero_body, 0)

    plsc.subcore_barrier()

    NH = K // 2

    for h in range(2):
        pltpu.sync_copy(srcr_hbm.at[pl.ds(w * K + h * NH, NH)], sidx)
        pltpu.sync_copy(dstr_hbm.at[pl.ds(w * K + h * NH, NH)], didx)
        pltpu.async_copy(h_hbm.at[sidx.at[0]], rows0, sem0)

        def pair_body(t, carry):
            e0 = 2 * t
            pltpu.make_async_copy(h_hbm.at[sidx.at[e0]], rows0, sem0).wait()
            pltpu.async_copy(h_hbm.at[sidx.at[e0 + 1]], rows1, sem1)
            pltpu.sync_copy(rows0, acc.at[didx.at[e0]], add=True)
            nxt = jnp.minimum(e0 + 2, NH - 1)
            pltpu.make_async_copy(h_hbm.at[sidx.at[e0 + 1]], rows1, sem1).wait()
            pltpu.async_copy(h_hbm.at[sidx.at[nxt]], rows0, sem0)
            pltpu.sync_copy(rows1, acc.at[didx.at[e0 + 1]], add=True)
            return carry

        lax.fori_loop(0, NH // 2, pair_body, 0)
        pltpu.make_async_copy(h_hbm.at[sidx.at[NH - 1]], rows0, sem0).wait()

    plsc.subcore_barrier()

    pltpu.sync_copy(
        acc.at[pl.ds(s * rows_per_tile, rows_per_tile)],
        out_hbm.at[pl.ds(c * NPAD + s * rows_per_tile, rows_per_tile)],
    )


def _make_sc_agg():
    mesh = plsc.VectorSubcoreMesh(
        core_axis_name="c", subcore_axis_name="s", num_cores=NC, num_subcores=NS
    )
    return pl.kernel(
        _sc_agg_body,
        out_type=jax.ShapeDtypeStruct((NC * NPAD, D), jnp.float32),
        mesh=mesh,
        scratch_types=[
            pltpu.VMEM((K // 2, CHUNK), jnp.int32),
            pltpu.VMEM((K // 2, CHUNK), jnp.int32),
            pltpu.VMEM((CHUNK, D), jnp.float32),
            pltpu.VMEM((CHUNK, D), jnp.float32),
            pltpu.VMEM((ZR, D), jnp.float32),
            pltpu.VMEM_SHARED((NPAD, D), jnp.float32),
            pltpu.SemaphoreType.DMA,
            pltpu.SemaphoreType.DMA,
        ],
    )


_SQRT_HALF = 0.7071067811865476


def _ffn_body(h_ref, p0_ref, p1_ref, w_ref, b_ref, o_ref):
    hsum = h_ref[...] + p0_ref[...] + p1_ref[...]
    y = jnp.dot(hsum, w_ref[...], preferred_element_type=jnp.float32) + b_ref[...]
    z = 0.5 * y * (1.0 + lax.erf(y * _SQRT_HALF))
    sq = jnp.sum(z * z, axis=-1, keepdims=True)
    o_ref[...] = z * lax.rsqrt(jnp.maximum(sq, 1e-12))


BR = 80


def _ffn(H, parts, Wp, bp):
    nblk = N // BR
    return pl.pallas_call(
        _ffn_body,
        out_shape=jax.ShapeDtypeStruct((N, D), jnp.float32),
        grid=(nblk,),
        in_specs=[
            pl.BlockSpec((BR, D), lambda i: (i, 0)),
            pl.BlockSpec((BR, D), lambda i: (i, 0)),
            pl.BlockSpec((BR, D), lambda i: (i + NPAD // BR, 0)),
            pl.BlockSpec((D, D), lambda i: (0, 0)),
            pl.BlockSpec((1, D), lambda i: (0, 0)),
        ],
        out_specs=pl.BlockSpec((BR, D), lambda i: (i, 0)),
    )(H, parts, parts, Wp, bp)


def kernel(H, edge_index, gamma, beta, moving_mean, moving_var, W, b):
    dst = edge_index[0].astype(jnp.int32)
    src = edge_index[1].astype(jnp.int32)
    pad = EPAD - E
    src_r = jnp.concatenate([src, jnp.zeros((pad,), jnp.int32)]).reshape(
        EPAD // CHUNK, CHUNK
    )
    padv = N + jnp.arange(pad, dtype=jnp.int32) % (NPAD - N)
    dst_r = jnp.concatenate([dst, padv]).reshape(EPAD // CHUNK, CHUNK)

    parts = _make_sc_agg()(H, src_r, dst_r)

    scale = gamma * lax.rsqrt(moving_var + BN_EPS)
    shift = beta - moving_mean * scale
    Wp = scale[:, None] * W
    bp = (shift @ W + b).reshape(1, D)

    return _ffn(H, parts, Wp, bp)


# speedup vs baseline: 2.9448x; 2.9448x over previous
import functools

import jax
import jax.numpy as jnp
from jax import lax
from jax.experimental import pallas as pl
from jax.experimental.pallas import tpu as pltpu
from jax.experimental.pallas import tpu_sc as plsc

N = 10000
E = 320000
D = 128
BN_EPS = 1e-3

NC = 2
NS = 16
NW = NC * NS
CHUNK = 128
K = 8 * (-(-E // (NW * CHUNK * 8)))
EPW = K * CHUNK
EPAD = NW * EPW
NPAD = 10240
ZR = 16


def _sc_agg_body(
    h_hbm, srcr_hbm, dstr_hbm, out_hbm, sidx, didx, rows0, rows1, zbuf, acc, sem0, sem1
):
    c = lax.axis_index("c")
    s = lax.axis_index("s")
    w = c * NS + s

    zeros16 = jnp.zeros((16,), jnp.float32)
    for r in range(ZR):
        for q in range(D // 16):
            zbuf[r, pl.ds(q * 16, 16)] = zeros16

    rows_per_tile = NPAD // NS

    def zero_body(t, carry):
        pltpu.sync_copy(zbuf, acc.at[pl.ds(s * rows_per_tile + t * ZR, ZR)])
        return carry

    lax.fori_loop(0, rows_per_tile // ZR, zero_body, 0)

    plsc.subcore_barrier()

    NH = K // 2

    for h in range(2):
        pltpu.sync_copy(srcr_hbm.at[pl.ds(w * K + h * NH, NH)], sidx)
        pltpu.sync_copy(dstr_hbm.at[pl.ds(w * K + h * NH, NH)], didx)
        pltpu.async_copy(h_hbm.at[sidx.at[0]], rows0, sem0)

        def pair_body(t, carry):
            e0 = 2 * t
            pltpu.make_async_copy(h_hbm.at[sidx.at[e0]], rows0, sem0).wait()
            pltpu.async_copy(h_hbm.at[sidx.at[e0 + 1]], rows1, sem1)
            pltpu.sync_copy(rows0, acc.at[didx.at[e0]], add=True)
            nxt = jnp.minimum(e0 + 2, NH - 1)
            pltpu.make_async_copy(h_hbm.at[sidx.at[e0 + 1]], rows1, sem1).wait()
            pltpu.async_copy(h_hbm.at[sidx.at[nxt]], rows0, sem0)
            pltpu.sync_copy(rows1, acc.at[didx.at[e0 + 1]], add=True)
            return carry

        lax.fori_loop(0, NH // 2, pair_body, 0)
        pltpu.make_async_copy(h_hbm.at[sidx.at[NH - 1]], rows0, sem0).wait()

    plsc.subcore_barrier()

    pltpu.sync_copy(
        acc.at[pl.ds(s * rows_per_tile, rows_per_tile)],
        out_hbm.at[pl.ds(c * NPAD + s * rows_per_tile, rows_per_tile)],
    )


def _make_sc_agg():
    mesh = plsc.VectorSubcoreMesh(
        core_axis_name="c", subcore_axis_name="s", num_cores=NC, num_subcores=NS
    )
    return pl.kernel(
        _sc_agg_body,
        out_type=jax.ShapeDtypeStruct((NC * NPAD, D), jnp.float32),
        mesh=mesh,
        scratch_types=[
            pltpu.VMEM((K // 2, CHUNK), jnp.int32),
            pltpu.VMEM((K // 2, CHUNK), jnp.int32),
            pltpu.VMEM((CHUNK, D), jnp.float32),
            pltpu.VMEM((CHUNK, D), jnp.float32),
            pltpu.VMEM((ZR, D), jnp.float32),
            pltpu.VMEM_SHARED((NPAD, D), jnp.float32),
            pltpu.SemaphoreType.DMA,
            pltpu.SemaphoreType.DMA,
        ],
    )


_SQRT_HALF = 0.7071067811865476


def _ffn_body(h_ref, p0_ref, p1_ref, w_ref, b_ref, o_ref):
    hsum = h_ref[...] + p0_ref[...] + p1_ref[...]
    y = jnp.dot(hsum, w_ref[...], preferred_element_type=jnp.float32) + b_ref[...]
    z = 0.5 * y * (1.0 + lax.erf(y * _SQRT_HALF))
    sq = jnp.sum(z * z, axis=-1, keepdims=True)
    o_ref[...] = z * lax.rsqrt(jnp.maximum(sq, 1e-12))


BR = 80


def _ffn(H, parts, Wp, bp):
    nblk = N // BR
    return pl.pallas_call(
        _ffn_body,
        out_shape=jax.ShapeDtypeStruct((N, D), jnp.float32),
        grid=(nblk,),
        in_specs=[
            pl.BlockSpec((BR, D), lambda i: (i, 0)),
            pl.BlockSpec((BR, D), lambda i: (i, 0)),
            pl.BlockSpec((BR, D), lambda i: (i + NPAD // BR, 0)),
            pl.BlockSpec((D, D), lambda i: (0, 0)),
            pl.BlockSpec((1, D), lambda i: (0, 0)),
        ],
        out_specs=pl.BlockSpec((BR, D), lambda i: (i, 0)),
    )(H, parts, parts, Wp, bp)


def kernel(H, edge_index, gamma, beta, moving_mean, moving_var, W, b):
    dst = edge_index[0].astype(jnp.int32)
    src = edge_index[1].astype(jnp.int32)
    pad = EPAD - E
    srcv = jnp.arange(pad, dtype=jnp.int32) % N
    src_r = jnp.concatenate([src, srcv]).reshape(EPAD // CHUNK, CHUNK)
    padv = N + jnp.arange(pad, dtype=jnp.int32) % (NPAD - N)
    dst_r = jnp.concatenate([dst, padv]).reshape(EPAD // CHUNK, CHUNK)

    parts = _make_sc_agg()(H, src_r, dst_r)

    scale = gamma * lax.rsqrt(moving_var + BN_EPS)
    shift = beta - moving_mean * scale
    Wp = scale[:, None] * W
    bp = (shift @ W + b).reshape(1, D)

    return _ffn(H, parts, Wp, bp)
